# Initial kernel scaffold; baseline (speedup 1.0000x reference)
import jax
import jax.numpy as jnp
from jax.experimental import pallas as pl


def kernel(centers, radii):
    raise NotImplementedError("write your pallas kernel here")


# speedup vs baseline: 1.5116x; 1.5116x over previous
import functools

import jax
import jax.numpy as jnp
import numpy as np
from jax.experimental import pallas as pl
from jax.experimental.pallas import tpu as pltpu

N = 200000
CAP = 4
GRID = 1024
L1 = 32
OVERSIZED = 64
SENTINEL = np.int32(2**31 - 1)
M = N * CAP * CAP * CAP

_BN = 1000


def _expand_bits_u32(x):
    x = (x | (x << 16)) & jnp.uint32(0x030000FF)
    x = (x | (x << 8)) & jnp.uint32(0x0300F00F)
    x = (x | (x << 4)) & jnp.uint32(0x030C30C3)
    x = (x | (x << 2)) & jnp.uint32(0x09249249)
    return x


def _enum_body(params_ref, c_ref, r_ref, morton_ref, counts_ref, over_ref):
    i = pl.program_id(0)
    gx0 = params_ref[0]
    gy0 = params_ref[1]
    gz0 = params_ref[2]
    vs = params_ref[3]
    r = r_ref[:, 0:1]
    gmin = []
    gmax = []
    for d, g0 in enumerate((gx0, gy0, gz0)):
        c = c_ref[:, d : d + 1]
        mn = c - r
        mx = c + r
        gmn = jnp.clip(jnp.floor((mn - g0) / vs).astype(jnp.int32), 0, GRID - 1)
        gmx = jnp.clip(jnp.floor((mx - g0) / vs).astype(jnp.int32), 0, GRID - 1)
        gmin.append(gmn)
        gmax.append(gmx)
    ext = [gmax[d] - gmin[d] + 1 for d in range(3)]
    numv = ext[0] * ext[1] * ext[2]
    oversized = (numv > OVERSIZED) | (ext[0] > CAP) | (ext[1] > CAP) | (ext[2] > CAP)
    counts_ref[:, 0:1] = jnp.where(oversized, 0, numv)
    sid = i * _BN + jax.lax.broadcasted_iota(jnp.int32, (_BN, 1), 0)
    over_ref[:, 0:1] = jnp.where(oversized, sid, -1)

    slot = jax.lax.broadcasted_iota(jnp.int32, (_BN, 64), 1)
    dx = slot >> 4
    dy = (slot >> 2) & 3
    dz = slot & 3
    gx = gmin[0] + dx
    gy = gmin[1] + dy
    gz = gmin[2] + dz
    valid = (dx < ext[0]) & (dy < ext[1]) & (dz < ext[2]) & jnp.logical_not(oversized)
    ex = _expand_bits_u32(jnp.clip(gx, 0, GRID - 1).astype(jnp.uint32))
    ey = _expand_bits_u32(jnp.clip(gy, 0, GRID - 1).astype(jnp.uint32))
    ez = _expand_bits_u32(jnp.clip(gz, 0, GRID - 1).astype(jnp.uint32))
    m = ((ex << 2) | (ey << 1) | ez).astype(jnp.int32)
    morton_ref[...] = jnp.where(valid, m, SENTINEL)


def _enumerate(centers, radii, global_min, voxel_size):
    params = jnp.concatenate([global_min, voxel_size[None]]).astype(jnp.float32)
    grid = N // _BN
    return pl.pallas_call(
        _enum_body,
        grid=(grid,),
        in_specs=[
            pl.BlockSpec(memory_space=pltpu.SMEM),
            pl.BlockSpec((_BN, 3), lambda i: (i, 0)),
            pl.BlockSpec((_BN, 1), lambda i: (i, 0)),
        ],
        out_specs=[
            pl.BlockSpec((_BN, 64), lambda i: (i, 0)),
            pl.BlockSpec((_BN, 1), lambda i: (i, 0)),
            pl.BlockSpec((_BN, 1), lambda i: (i, 0)),
        ],
        out_shape=[
            jax.ShapeDtypeStruct((N, 64), jnp.int32),
            jax.ShapeDtypeStruct((N, 1), jnp.int32),
            jax.ShapeDtypeStruct((N, 1), jnp.int32),
        ],
    )(params, centers, radii.reshape(N, 1))


def _interleave15():
    r = np.arange(L1 * L1 * L1, dtype=np.uint32)
    rx, ry, rz = r >> 10, (r >> 5) & 31, r & 31

    def exp5(v):
        v = (v | (v << 8)) & np.uint32(0x0300F00F)
        v = (v | (v << 4)) & np.uint32(0x030C30C3)
        v = (v | (v << 2)) & np.uint32(0x09249249)
        return v

    return ((exp5(rx) << 2) | (exp5(ry) << 1) | exp5(rz)).astype(np.int32)


_INTERLEAVE15 = _interleave15()


def kernel(centers, radii):
    min_corners = centers - radii[:, None]
    max_corners = centers + radii[:, None]
    lo = jnp.quantile(min_corners, 0.01, axis=0)
    hi = jnp.quantile(max_corners, 0.99, axis=0)
    pad = 0.1 * (hi - lo)
    global_min = lo - pad
    extents = max_corners - min_corners
    short_axis = jnp.min(extents, axis=1)
    voxel_size = 2.0 * jnp.median(short_axis) + 1e-7

    morton2d, counts2d, over2d = _enumerate(centers, radii, global_min, voxel_size)
    counts = counts2d[:, 0]
    oversized_sphere_ids = over2d[:, 0]
    m_flat = morton2d.reshape(-1)
    v_flat = m_flat != SENTINEL
    s_flat = jnp.repeat(jnp.arange(N, dtype=jnp.int32), 64, total_repeat_length=M)

    offsets = jnp.concatenate(
        [jnp.zeros((1,), jnp.int32), jnp.cumsum(counts).astype(jnp.int32)]
    )
    total_pairs = jnp.sum(counts)

    order = jnp.argsort(m_flat)
    pairs_morton = m_flat[order]
    pairs_sphere_id = s_flat[order]
    valid_sorted = v_flat[order]

    histm = jnp.zeros((L1 * L1 * L1,), jnp.int32).at[m_flat >> 15].add(
        v_flat.astype(jnp.int32), mode="drop"
    )
    l1_counts = histm[jnp.asarray(_INTERLEAVE15)]
    csum = jnp.cumsum(l1_counts).astype(jnp.int32) - l1_counts
    l1_offsets = jnp.where(l1_counts > 0, csum, -1).astype(jnp.int32)

    prev = jnp.concatenate([jnp.full((1,), -1, jnp.int32), pairs_morton[:-1]])
    num_unique = jnp.sum(((pairs_morton != prev) & valid_sorted).astype(jnp.int32))

    return (pairs_morton, pairs_sphere_id, l1_offsets, counts, offsets,
            total_pairs, num_unique, oversized_sphere_ids, global_min, voxel_size)
